# baseline (speedup 1.0000x reference)
import functools

import jax
import jax.numpy as jnp
from jax import lax
from jax.experimental import pallas as pl
from jax.experimental.pallas import tpu as pltpu
from jax.experimental.pallas import tpu_sc as plsc

DIM = 64
NB = 16384
SEQ = 100
VOC = 1000000
NCORE = 2
NSUB = 16
NWORK = NCORE * NSUB
DPW = NB // NWORK
LANES = 16
NCH = DIM // LANES
GK = 4
HALF = DPW // 2
NGRP = HALF // GK
RUN = 4

NV = 512
VSPLIT = 500224
VOC2 = 2 * VSPLIT


def _tc_relayout(table_t):
    def body(lo_ref, hi_ref, o_ref):
        lo = jnp.swapaxes(lo_ref[...], 0, 1)
        hi = jnp.swapaxes(hi_ref[...], 0, 1)
        o_ref[...] = jnp.concatenate([lo, hi], axis=1)

    return pl.pallas_call(
        body,
        grid=(VSPLIT // NV,),
        in_specs=[
            pl.BlockSpec((DIM, NV), lambda i: (0, i)),
            pl.BlockSpec((DIM, NV), lambda i: (0, i + VSPLIT // NV)),
        ],
        out_specs=pl.BlockSpec((NV, 2 * DIM), lambda i: (i, 0)),
        out_shape=jax.ShapeDtypeStruct((VSPLIT, 2 * DIM), jnp.float32),
    )(table_t, table_t)


def _sc_pool(token_ids, table_lin):
    mesh = plsc.VectorSubcoreMesh(core_axis_name="c", subcore_axis_name="s")

    @functools.partial(
        pl.kernel,
        out_type=jax.ShapeDtypeStruct((NB, DIM), jnp.float32),
        mesh=mesh,
        scratch_types=[
            pltpu.VMEM((HALF, SEQ), jnp.int32),
            pltpu.VMEM((2 * GK, SEQ, DIM), jnp.float32),
            pltpu.VMEM((DPW, DIM), jnp.float32),
            pltpu.SemaphoreType.DMA,
            pltpu.SemaphoreType.DMA,
        ],
        compiler_params=pltpu.CompilerParams(use_tc_tiling_on_sc=False),
    )
    def pool(tok_hbm, table_hbm, out_hbm, idx_v, rows_v, acc_v, sem_a, sem_b):
        wid = lax.axis_index("s") * NCORE + lax.axis_index("c")
        base = wid * DPW

        def fire(g, slot0, sem):
            for i in range(GK):
                pltpu.async_copy(
                    table_hbm.at[idx_v.at[g * GK + i]], rows_v.at[slot0 + i], sem
                )

        def drain(g, slot0, sem):
            for i in range(GK):
                pltpu.make_async_copy(
                    table_hbm.at[idx_v.at[g * GK + i]], rows_v.at[slot0 + i], sem
                ).wait()

        def reduce_group(g, slot0, acc_base):
            for i in range(GK):
                slot = slot0 + i

                def red(r, accs, slot=slot):
                    out = list(accs)
                    for rr in range(RUN):
                        row = r * RUN + rr
                        for c in range(NCH):
                            out[c] = out[c] + rows_v[
                                slot, row, pl.ds(c * LANES, LANES)
                            ]
                    return tuple(out)

                accs = lax.fori_loop(
                    0, SEQ // RUN, red,
                    tuple(jnp.zeros((LANES,), jnp.float32) for _ in range(NCH)),
                )
                for c in range(NCH):
                    acc_v[acc_base + g * GK + i, pl.ds(c * LANES, LANES)] = accs[c]

        for h in range(2):
            hbase = base + h * HALF
            pltpu.sync_copy(tok_hbm.at[pl.ds(hbase, HALF), :], idx_v)
            fire(0, 0, sem_a)

            def jj_body(jj, carry, h=h):
                g = 2 * jj
                fire(g + 1, GK, sem_b)
                drain(g, 0, sem_a)
                reduce_group(g, 0, h * HALF)

                @pl.when(g + 2 < NGRP)
                def _():
                    fire(g + 2, 0, sem_a)

                drain(g + 1, GK, sem_b)
                reduce_group(g + 1, GK, h * HALF)
                return carry

            lax.fori_loop(0, NGRP // 2, jj_body, 0)

        pltpu.sync_copy(acc_v, out_hbm.at[pl.ds(base, DPW), :])

    return pool(token_ids, table_lin.reshape(VOC2, DIM))


def _tc_proj(sums, W, b):
    blk = 2048

    def proj(s_ref, w_ref, b_ref, o_ref):
        o_ref[...] = (
            lax.dot_general(
                s_ref[...], w_ref[...], (((1,), (1,)), ((), ())),
                preferred_element_type=jnp.float32,
            ) * (1.0 / SEQ)
            + b_ref[...]
        )

    return pl.pallas_call(
        proj,
        grid=(NB // blk,),
        in_specs=[
            pl.BlockSpec((blk, DIM), lambda i: (i, 0)),
            pl.BlockSpec((DIM, DIM), lambda i: (0, 0)),
            pl.BlockSpec((1, DIM), lambda i: (0, 0)),
        ],
        out_specs=pl.BlockSpec((blk, DIM), lambda i: (i, 0)),
        out_shape=jax.ShapeDtypeStruct((NB, DIM), jnp.float32),
    )(sums, W, b.reshape(1, DIM))


@jax.jit
def kernel(token_ids, table, W, b):
    table_t = jnp.swapaxes(table, 0, 1)
    table_pairs = _tc_relayout(table_t)
    tok2 = jnp.where(token_ids < VSPLIT, 2 * token_ids,
                     2 * token_ids - (VOC2 - 1))
    sums = _sc_pool(tok2, table_pairs.reshape(VOC2 * DIM))
    return _tc_proj(sums, W, b)


# speedup vs baseline: 1.6453x; 1.6453x over previous
import functools

import jax
import jax.numpy as jnp
from jax import lax
from jax.experimental import pallas as pl
from jax.experimental.pallas import tpu as pltpu
from jax.experimental.pallas import tpu_sc as plsc

DIM = 64
NB = 16384
SEQ = 100
VOC = 1000000
NCORE = 2
NSUB = 16
NWORK = NCORE * NSUB
DPW = NB // NWORK
LANES = 16
NCH = DIM // LANES
GK = 4
HALF = DPW // 2
NGRP = HALF // GK
RUN = 4

NV = 2048
VSPLIT = 501760
VOC2 = 2 * VSPLIT


def _tc_relayout(table_t):
    def body(lo_ref, hi_ref, o_ref):
        lo = jnp.swapaxes(lo_ref[...], 0, 1)
        hi = jnp.swapaxes(hi_ref[...], 0, 1)
        o_ref[...] = jnp.concatenate([lo, hi], axis=1)

    return pl.pallas_call(
        body,
        grid=(VSPLIT // NV,),
        in_specs=[
            pl.BlockSpec((DIM, NV), lambda i: (0, i)),
            pl.BlockSpec(
                (DIM, NV),
                lambda i: (0, jnp.minimum(i + VSPLIT // NV, (VOC - 1) // NV)),
            ),
        ],
        out_specs=pl.BlockSpec((NV, 2 * DIM), lambda i: (i, 0)),
        out_shape=jax.ShapeDtypeStruct((VSPLIT, 2 * DIM), jnp.float32),
    )(table_t, table_t)


def _sc_pool(token_ids, table_lin):
    mesh = plsc.VectorSubcoreMesh(core_axis_name="c", subcore_axis_name="s")

    @functools.partial(
        pl.kernel,
        out_type=jax.ShapeDtypeStruct((NB, DIM), jnp.float32),
        mesh=mesh,
        scratch_types=[
            pltpu.VMEM((HALF, SEQ), jnp.int32),
            pltpu.VMEM((2 * GK, SEQ, DIM), jnp.float32),
            pltpu.VMEM((DPW, DIM), jnp.float32),
            pltpu.SemaphoreType.DMA,
            pltpu.SemaphoreType.DMA,
        ],
        compiler_params=pltpu.CompilerParams(use_tc_tiling_on_sc=False),
    )
    def pool(tok_hbm, table_hbm, out_hbm, idx_v, rows_v, acc_v, sem_a, sem_b):
        wid = lax.axis_index("s") * NCORE + lax.axis_index("c")
        base = wid * DPW

        def fire(g, slot0, sem):
            for i in range(GK):
                pltpu.async_copy(
                    table_hbm.at[idx_v.at[g * GK + i]], rows_v.at[slot0 + i], sem
                )

        def drain(g, slot0, sem):
            for i in range(GK):
                pltpu.make_async_copy(
                    table_hbm.at[idx_v.at[g * GK + i]], rows_v.at[slot0 + i], sem
                ).wait()

        def reduce_group(g, slot0, acc_base):
            for i in range(GK):
                slot = slot0 + i

                def red(r, accs, slot=slot):
                    out = list(accs)
                    for rr in range(RUN):
                        row = r * RUN + rr
                        for c in range(NCH):
                            out[c] = out[c] + rows_v[
                                slot, row, pl.ds(c * LANES, LANES)
                            ]
                    return tuple(out)

                accs = lax.fori_loop(
                    0, SEQ // RUN, red,
                    tuple(jnp.zeros((LANES,), jnp.float32) for _ in range(NCH)),
                )
                for c in range(NCH):
                    acc_v[acc_base + g * GK + i, pl.ds(c * LANES, LANES)] = accs[c]

        for h in range(2):
            hbase = base + h * HALF
            pltpu.sync_copy(tok_hbm.at[pl.ds(hbase, HALF), :], idx_v)
            fire(0, 0, sem_a)

            def jj_body(jj, carry, h=h):
                g = 2 * jj
                fire(g + 1, GK, sem_b)
                drain(g, 0, sem_a)
                reduce_group(g, 0, h * HALF)

                @pl.when(g + 2 < NGRP)
                def _():
                    fire(g + 2, 0, sem_a)

                drain(g + 1, GK, sem_b)
                reduce_group(g + 1, GK, h * HALF)
                return carry

            lax.fori_loop(0, NGRP // 2, jj_body, 0)

        pltpu.sync_copy(acc_v, out_hbm.at[pl.ds(base, DPW), :])

    return pool(token_ids, table_lin.reshape(VOC2, DIM))


def _tc_proj(sums, W, b):
    blk = 2048

    def proj(s_ref, w_ref, b_ref, o_ref):
        o_ref[...] = (
            lax.dot_general(
                s_ref[...], w_ref[...], (((1,), (1,)), ((), ())),
                preferred_element_type=jnp.float32,
            ) * (1.0 / SEQ)
            + b_ref[...]
        )

    return pl.pallas_call(
        proj,
        grid=(NB // blk,),
        in_specs=[
            pl.BlockSpec((blk, DIM), lambda i: (i, 0)),
            pl.BlockSpec((DIM, DIM), lambda i: (0, 0)),
            pl.BlockSpec((1, DIM), lambda i: (0, 0)),
        ],
        out_specs=pl.BlockSpec((blk, DIM), lambda i: (i, 0)),
        out_shape=jax.ShapeDtypeStruct((NB, DIM), jnp.float32),
    )(sums, W, b.reshape(1, DIM))


@jax.jit
def kernel(token_ids, table, W, b):
    table_t = jnp.swapaxes(table, 0, 1)
    table_pairs = _tc_relayout(table_t)
    tok2 = jnp.where(token_ids < VSPLIT, 2 * token_ids,
                     2 * token_ids - (VOC2 - 1))
    sums = _sc_pool(tok2, table_pairs.reshape(VOC2 * DIM))
    return _tc_proj(sums, W, b)


# speedup vs baseline: 1.9978x; 1.2143x over previous
import functools

import jax
import jax.numpy as jnp
from jax import lax
from jax.experimental import pallas as pl
from jax.experimental.pallas import tpu as pltpu
from jax.experimental.pallas import tpu_sc as plsc

DIM = 64
NB = 16384
SEQ = 100
VOC = 1000000
NCORE = 2
NSUB = 16
NWORK = NCORE * NSUB
DPW = NB // NWORK
LANES = 16
NCH = DIM // LANES
GK = 4
HALF = DPW // 2
NGRP = HALF // GK
RUN = 4

NV = 8192
VSPLIT = 507904
VOC2 = 2 * VSPLIT


def _tc_relayout(table_t):
    def body(lo_ref, hi_ref, o_ref):
        lo = jnp.swapaxes(lo_ref[...], 0, 1)
        hi = jnp.swapaxes(hi_ref[...], 0, 1)
        o_ref[...] = jnp.concatenate([lo, hi], axis=1)

    return pl.pallas_call(
        body,
        grid=(VSPLIT // NV,),
        in_specs=[
            pl.BlockSpec((DIM, NV), lambda i: (0, i)),
            pl.BlockSpec(
                (DIM, NV),
                lambda i: (0, jnp.minimum(i + VSPLIT // NV, (VOC - 1) // NV)),
            ),
        ],
        out_specs=pl.BlockSpec((NV, 2 * DIM), lambda i: (i, 0)),
        out_shape=jax.ShapeDtypeStruct((VSPLIT, 2 * DIM), jnp.float32),
    )(table_t, table_t)


def _sc_pool(token_ids, table_lin):
    mesh = plsc.VectorSubcoreMesh(core_axis_name="c", subcore_axis_name="s")

    @functools.partial(
        pl.kernel,
        out_type=jax.ShapeDtypeStruct((NB, DIM), jnp.float32),
        mesh=mesh,
        scratch_types=[
            pltpu.VMEM((HALF, SEQ), jnp.int32),
            pltpu.VMEM((2 * GK, SEQ, DIM), jnp.float32),
            pltpu.VMEM((DPW, DIM), jnp.float32),
            pltpu.SemaphoreType.DMA,
            pltpu.SemaphoreType.DMA,
        ],
        compiler_params=pltpu.CompilerParams(use_tc_tiling_on_sc=False),
    )
    def pool(tok_hbm, table_hbm, out_hbm, idx_v, rows_v, acc_v, sem_a, sem_b):
        wid = lax.axis_index("s") * NCORE + lax.axis_index("c")
        base = wid * DPW

        def fire(g, slot0, sem):
            for i in range(GK):
                pltpu.async_copy(
                    table_hbm.at[idx_v.at[g * GK + i]], rows_v.at[slot0 + i], sem
                )

        def drain(g, slot0, sem):
            for i in range(GK):
                pltpu.make_async_copy(
                    table_hbm.at[idx_v.at[g * GK + i]], rows_v.at[slot0 + i], sem
                ).wait()

        def reduce_group(g, slot0, acc_base):
            for i in range(GK):
                slot = slot0 + i

                def red(r, accs, slot=slot):
                    out = list(accs)
                    for rr in range(RUN):
                        row = r * RUN + rr
                        for c in range(NCH):
                            out[c] = out[c] + rows_v[
                                slot, row, pl.ds(c * LANES, LANES)
                            ]
                    return tuple(out)

                accs = lax.fori_loop(
                    0, SEQ // RUN, red,
                    tuple(jnp.zeros((LANES,), jnp.float32) for _ in range(NCH)),
                )
                for c in range(NCH):
                    acc_v[acc_base + g * GK + i, pl.ds(c * LANES, LANES)] = accs[c]

        for h in range(2):
            hbase = base + h * HALF
            pltpu.sync_copy(tok_hbm.at[pl.ds(hbase, HALF), :], idx_v)
            fire(0, 0, sem_a)

            def jj_body(jj, carry, h=h):
                g = 2 * jj
                fire(g + 1, GK, sem_b)
                drain(g, 0, sem_a)
                reduce_group(g, 0, h * HALF)

                @pl.when(g + 2 < NGRP)
                def _():
                    fire(g + 2, 0, sem_a)

                drain(g + 1, GK, sem_b)
                reduce_group(g + 1, GK, h * HALF)
                return carry

            lax.fori_loop(0, NGRP // 2, jj_body, 0)

        pltpu.sync_copy(acc_v, out_hbm.at[pl.ds(base, DPW), :])

    return pool(token_ids, table_lin.reshape(VOC2, DIM))


def _tc_proj(sums, W, b):
    blk = 2048

    def proj(s_ref, w_ref, b_ref, o_ref):
        o_ref[...] = (
            lax.dot_general(
                s_ref[...], w_ref[...], (((1,), (1,)), ((), ())),
                preferred_element_type=jnp.float32,
            ) * (1.0 / SEQ)
            + b_ref[...]
        )

    return pl.pallas_call(
        proj,
        grid=(NB // blk,),
        in_specs=[
            pl.BlockSpec((blk, DIM), lambda i: (i, 0)),
            pl.BlockSpec((DIM, DIM), lambda i: (0, 0)),
            pl.BlockSpec((1, DIM), lambda i: (0, 0)),
        ],
        out_specs=pl.BlockSpec((blk, DIM), lambda i: (i, 0)),
        out_shape=jax.ShapeDtypeStruct((NB, DIM), jnp.float32),
    )(sums, W, b.reshape(1, DIM))


@jax.jit
def kernel(token_ids, table, W, b):
    table_t = jnp.swapaxes(table, 0, 1)
    table_pairs = _tc_relayout(table_t)
    tok2 = jnp.where(token_ids < VSPLIT, 2 * token_ids,
                     2 * token_ids - (VOC2 - 1))
    sums = _sc_pool(tok2, table_pairs.reshape(VOC2 * DIM))
    return _tc_proj(sums, W, b)


# speedup vs baseline: 2.0668x; 1.0345x over previous
import functools

import jax
import jax.numpy as jnp
from jax import lax
from jax.experimental import pallas as pl
from jax.experimental.pallas import tpu as pltpu
from jax.experimental.pallas import tpu_sc as plsc

DIM = 64
NB = 16384
SEQ = 100
VOC = 1000000
NCORE = 2
NSUB = 16
NWORK = NCORE * NSUB
DPW = NB // NWORK
LANES = 16
NCH = DIM // LANES
GK = 4
HALF = DPW // 2
NGRP = HALF // GK
RUN = 4

NV = 16384
VSPLIT = 507904
VOC2 = 2 * VSPLIT


def _tc_relayout(table_t):
    def body(lo_ref, hi_ref, o_ref):
        lo = jnp.swapaxes(lo_ref[...], 0, 1)
        hi = jnp.swapaxes(hi_ref[...], 0, 1)
        o_ref[...] = jnp.concatenate([lo, hi], axis=1)

    return pl.pallas_call(
        body,
        grid=(VSPLIT // NV,),
        in_specs=[
            pl.BlockSpec((DIM, NV), lambda i: (0, i)),
            pl.BlockSpec(
                (DIM, NV),
                lambda i: (0, jnp.minimum(i + VSPLIT // NV, (VOC - 1) // NV)),
            ),
        ],
        out_specs=pl.BlockSpec((NV, 2 * DIM), lambda i: (i, 0)),
        out_shape=jax.ShapeDtypeStruct((VSPLIT, 2 * DIM), jnp.float32),
    )(table_t, table_t)


def _sc_pool(token_ids, table_lin):
    mesh = plsc.VectorSubcoreMesh(core_axis_name="c", subcore_axis_name="s")

    @functools.partial(
        pl.kernel,
        out_type=jax.ShapeDtypeStruct((NB, DIM), jnp.float32),
        mesh=mesh,
        scratch_types=[
            pltpu.VMEM((HALF, SEQ), jnp.int32),
            pltpu.VMEM((2 * GK, SEQ, DIM), jnp.float32),
            pltpu.VMEM((DPW, DIM), jnp.float32),
            pltpu.SemaphoreType.DMA,
            pltpu.SemaphoreType.DMA,
        ],
        compiler_params=pltpu.CompilerParams(use_tc_tiling_on_sc=False),
    )
    def pool(tok_hbm, table_hbm, out_hbm, idx_v, rows_v, acc_v, sem_a, sem_b):
        wid = lax.axis_index("s") * NCORE + lax.axis_index("c")
        base = wid * DPW

        def fire(g, slot0, sem):
            for i in range(GK):
                pltpu.async_copy(
                    table_hbm.at[idx_v.at[g * GK + i]], rows_v.at[slot0 + i], sem
                )

        def drain(g, slot0, sem):
            for i in range(GK):
                pltpu.make_async_copy(
                    table_hbm.at[idx_v.at[g * GK + i]], rows_v.at[slot0 + i], sem
                ).wait()

        def reduce_group(g, slot0, acc_base):
            for i in range(GK):
                slot = slot0 + i

                def red(r, accs, slot=slot):
                    out = list(accs)
                    for rr in range(RUN):
                        row = r * RUN + rr
                        for c in range(NCH):
                            out[c] = out[c] + rows_v[
                                slot, row, pl.ds(c * LANES, LANES)
                            ]
                    return tuple(out)

                accs = lax.fori_loop(
                    0, SEQ // RUN, red,
                    tuple(jnp.zeros((LANES,), jnp.float32) for _ in range(NCH)),
                )
                for c in range(NCH):
                    acc_v[acc_base + g * GK + i, pl.ds(c * LANES, LANES)] = accs[c]

        for h in range(2):
            hbase = base + h * HALF
            pltpu.sync_copy(tok_hbm.at[pl.ds(hbase, HALF), :], idx_v)
            fire(0, 0, sem_a)

            def jj_body(jj, carry, h=h):
                g = 2 * jj
                fire(g + 1, GK, sem_b)
                drain(g, 0, sem_a)
                reduce_group(g, 0, h * HALF)

                @pl.when(g + 2 < NGRP)
                def _():
                    fire(g + 2, 0, sem_a)

                drain(g + 1, GK, sem_b)
                reduce_group(g + 1, GK, h * HALF)
                return carry

            lax.fori_loop(0, NGRP // 2, jj_body, 0)

        pltpu.sync_copy(acc_v, out_hbm.at[pl.ds(base, DPW), :])

    return pool(token_ids, table_lin.reshape(VOC2, DIM))


def _tc_proj(sums, W, b):
    blk = 2048

    def proj(s_ref, w_ref, b_ref, o_ref):
        o_ref[...] = (
            lax.dot_general(
                s_ref[...], w_ref[...], (((1,), (1,)), ((), ())),
                preferred_element_type=jnp.float32,
            ) * (1.0 / SEQ)
            + b_ref[...]
        )

    return pl.pallas_call(
        proj,
        grid=(NB // blk,),
        in_specs=[
            pl.BlockSpec((blk, DIM), lambda i: (i, 0)),
            pl.BlockSpec((DIM, DIM), lambda i: (0, 0)),
            pl.BlockSpec((1, DIM), lambda i: (0, 0)),
        ],
        out_specs=pl.BlockSpec((blk, DIM), lambda i: (i, 0)),
        out_shape=jax.ShapeDtypeStruct((NB, DIM), jnp.float32),
    )(sums, W, b.reshape(1, DIM))


@jax.jit
def kernel(token_ids, table, W, b):
    table_t = jnp.swapaxes(table, 0, 1)
    table_pairs = _tc_relayout(table_t)
    tok2 = jnp.where(token_ids < VSPLIT, 2 * token_ids,
                     2 * token_ids - (VOC2 - 1))
    sums = _sc_pool(tok2, table_pairs.reshape(VOC2 * DIM))
    return _tc_proj(sums, W, b)


# speedup vs baseline: 2.0950x; 1.0136x over previous
import functools

import jax
import jax.numpy as jnp
from jax import lax
from jax.experimental import pallas as pl
from jax.experimental.pallas import tpu as pltpu
from jax.experimental.pallas import tpu_sc as plsc

DIM = 64
NB = 16384
SEQ = 100
VOC = 1000000
NCORE = 2
NSUB = 16
NWORK = NCORE * NSUB
DPW = NB // NWORK
LANES = 16
NCH = DIM // LANES
GK = 8
QUART = DPW // 4
NG = QUART // GK
RUN = 4

NV = 16384
VSPLIT = 507904
VOC2 = 2 * VSPLIT


def _tc_relayout(table_t):
    def body(lo_ref, hi_ref, o_ref):
        lo = jnp.swapaxes(lo_ref[...], 0, 1)
        hi = jnp.swapaxes(hi_ref[...], 0, 1)
        o_ref[...] = jnp.concatenate([lo, hi], axis=1)

    return pl.pallas_call(
        body,
        grid=(VSPLIT // NV,),
        in_specs=[
            pl.BlockSpec((DIM, NV), lambda i: (0, i)),
            pl.BlockSpec(
                (DIM, NV),
                lambda i: (0, jnp.minimum(i + VSPLIT // NV, (VOC - 1) // NV)),
            ),
        ],
        out_specs=pl.BlockSpec((NV, 2 * DIM), lambda i: (i, 0)),
        out_shape=jax.ShapeDtypeStruct((VSPLIT, 2 * DIM), jnp.float32),
    )(table_t, table_t)


def _sc_pool(tok2, table_lin):
    mesh = plsc.VectorSubcoreMesh(core_axis_name="c", subcore_axis_name="s")

    @functools.partial(
        pl.kernel,
        out_type=jax.ShapeDtypeStruct((NB, DIM), jnp.float32),
        mesh=mesh,
        scratch_types=[
            pltpu.VMEM((QUART, SEQ), jnp.int32),
            pltpu.VMEM((2 * GK, SEQ, DIM), jnp.float32),
            pltpu.VMEM((2 * GK, DIM), jnp.float32),
            pltpu.SemaphoreType.DMA,
            pltpu.SemaphoreType.DMA,
            pltpu.SemaphoreType.DMA,
            pltpu.SemaphoreType.DMA,
        ],
        compiler_params=pltpu.CompilerParams(use_tc_tiling_on_sc=False),
    )
    def pool(tok_hbm, table_hbm, out_hbm, idx_v, rows_v, st_v,
             sem_a, sem_b, soa, sob):
        wid = lax.axis_index("s") * NCORE + lax.axis_index("c")
        base = wid * DPW

        def fire(g, slot0, sem):
            for i in range(GK):
                pltpu.async_copy(
                    table_hbm.at[idx_v.at[g * GK + i]], rows_v.at[slot0 + i],
                    sem)

        def drain(g, slot0, sem):
            for i in range(GK):
                pltpu.make_async_copy(
                    table_hbm.at[idx_v.at[g * GK + i]], rows_v.at[slot0 + i],
                    sem).wait()

        def start_out(qbase, g, slot0, sem):
            pltpu.async_copy(
                st_v.at[pl.ds(slot0, GK)],
                out_hbm.at[pl.ds(qbase + g * GK, GK), :], sem)

        def wait_out(qbase, g, slot0, sem):
            pltpu.make_async_copy(
                st_v.at[pl.ds(slot0, GK)],
                out_hbm.at[pl.ds(qbase + g * GK, GK), :], sem).wait()

        def reduce_group(slot0):
            for i in range(GK):
                slot = slot0 + i

                def red(r, accs, slot=slot):
                    out = list(accs)
                    for rr in range(RUN):
                        row = r * RUN + rr
                        for c in range(NCH):
                            out[c] = out[c] + rows_v[
                                slot, row, pl.ds(c * LANES, LANES)
                            ]
                    return tuple(out)

                accs = lax.fori_loop(
                    0, SEQ // RUN, red,
                    tuple(jnp.zeros((LANES,), jnp.float32) for _ in range(NCH)),
                )
                for c in range(NCH):
                    st_v[slot, pl.ds(c * LANES, LANES)] = accs[c]

        for q in range(DPW // QUART):
            qbase = base + q * QUART
            pltpu.sync_copy(tok_hbm.at[pl.ds(qbase, QUART), :], idx_v)
            fire(0, 0, sem_a)

            def jj_body(jj, carry, q=q, qbase=qbase):
                g = 2 * jj
                fire(g + 1, GK, sem_b)
                drain(g, 0, sem_a)

                @pl.when(jj > 0 if q == 0 else jj >= 0)
                def _():
                    wait_out(qbase, g, 0, soa)

                reduce_group(0)
                start_out(qbase, g, 0, soa)

                @pl.when(g + 2 < NG)
                def _():
                    fire(g + 2, 0, sem_a)

                drain(g + 1, GK, sem_b)

                @pl.when(jj > 0 if q == 0 else jj >= 0)
                def _():
                    wait_out(qbase, g + 1, GK, sob)

                reduce_group(GK)
                start_out(qbase, g + 1, GK, sob)
                return carry

            lax.fori_loop(0, NG // 2, jj_body, 0)

        wait_out(base, 0, 0, soa)
        wait_out(base, 0, GK, sob)

    return pool(tok2, table_lin.reshape(VOC2, DIM))


def _tc_proj(sums, W, b):
    blk = 2048

    def proj(s_ref, w_ref, b_ref, o_ref):
        o_ref[...] = (
            lax.dot_general(
                s_ref[...], w_ref[...], (((1,), (1,)), ((), ())),
                preferred_element_type=jnp.float32,
            ) * (1.0 / SEQ)
            + b_ref[...]
        )

    return pl.pallas_call(
        proj,
        grid=(NB // blk,),
        in_specs=[
            pl.BlockSpec((blk, DIM), lambda i: (i, 0)),
            pl.BlockSpec((DIM, DIM), lambda i: (0, 0)),
            pl.BlockSpec((1, DIM), lambda i: (0, 0)),
        ],
        out_specs=pl.BlockSpec((blk, DIM), lambda i: (i, 0)),
        out_shape=jax.ShapeDtypeStruct((NB, DIM), jnp.float32),
    )(sums, W, b.reshape(1, DIM))


@jax.jit
def kernel(token_ids, table, W, b):
    table_t = jnp.swapaxes(table, 0, 1)
    table_pairs = _tc_relayout(table_t)
    tok2 = jnp.where(token_ids < VSPLIT, 2 * token_ids,
                     2 * token_ids - (VOC2 - 1))
    sums = _sc_pool(tok2, table_pairs.reshape(VOC2 * DIM))
    return _tc_proj(sums, W, b)


# speedup vs baseline: 2.2681x; 1.0827x over previous
import functools

import jax
import jax.numpy as jnp
from jax import lax
from jax.experimental import pallas as pl
from jax.experimental.pallas import tpu as pltpu
from jax.experimental.pallas import tpu_sc as plsc

DIM = 64
NB = 16384
SEQ = 100
VOC = 1000000
NCORE = 2
NSUB = 16
NWORK = NCORE * NSUB
DPW = NB // NWORK
LANES = 16
NCH = DIM // LANES
GK = 8
QUART = DPW // 4
NG = QUART // GK
RUN = 4

NV = 16384
VSPLIT = 507904
VOC2 = 2 * VSPLIT


def _tc_relayout(table_t):
    def body(lo_ref, hi_ref, ilo_ref, ihi_ref, o_ref):
        dn = (((0,), (0,)), ((), ()))
        o_ref[...] = lax.dot_general(
            lo_ref[...], ilo_ref[...], dn, preferred_element_type=jnp.float32
        ) + lax.dot_general(
            hi_ref[...], ihi_ref[...], dn, preferred_element_type=jnp.float32
        )

    eye = jnp.eye(DIM, dtype=jnp.float32)
    zero = jnp.zeros((DIM, DIM), jnp.float32)
    ilo = jnp.concatenate([eye, zero], axis=1)
    ihi = jnp.concatenate([zero, eye], axis=1)
    return pl.pallas_call(
        body,
        grid=(VSPLIT // NV,),
        in_specs=[
            pl.BlockSpec((DIM, NV), lambda i: (0, i)),
            pl.BlockSpec(
                (DIM, NV),
                lambda i: (0, jnp.minimum(i + VSPLIT // NV, (VOC - 1) // NV)),
            ),
            pl.BlockSpec((DIM, 2 * DIM), lambda i: (0, 0)),
            pl.BlockSpec((DIM, 2 * DIM), lambda i: (0, 0)),
        ],
        out_specs=pl.BlockSpec((NV, 2 * DIM), lambda i: (i, 0)),
        out_shape=jax.ShapeDtypeStruct((VSPLIT, 2 * DIM), jnp.float32),
    )(table_t, table_t, ilo, ihi)


def _sc_pool(tok2, table_lin):
    mesh = plsc.VectorSubcoreMesh(core_axis_name="c", subcore_axis_name="s")

    @functools.partial(
        pl.kernel,
        out_type=jax.ShapeDtypeStruct((NB, DIM), jnp.float32),
        mesh=mesh,
        scratch_types=[
            pltpu.VMEM((QUART, SEQ), jnp.int32),
            pltpu.VMEM((2 * GK, SEQ, DIM), jnp.float32),
            pltpu.VMEM((2 * GK, DIM), jnp.float32),
            pltpu.SemaphoreType.DMA,
            pltpu.SemaphoreType.DMA,
            pltpu.SemaphoreType.DMA,
            pltpu.SemaphoreType.DMA,
        ],
        compiler_params=pltpu.CompilerParams(use_tc_tiling_on_sc=False),
    )
    def pool(tok_hbm, table_hbm, out_hbm, idx_v, rows_v, st_v,
             sem_a, sem_b, soa, sob):
        wid = lax.axis_index("s") * NCORE + lax.axis_index("c")
        base = wid * DPW

        def fire(g, slot0, sem):
            for i in range(GK):
                pltpu.async_copy(
                    table_hbm.at[idx_v.at[g * GK + i]], rows_v.at[slot0 + i],
                    sem)

        def drain(g, slot0, sem):
            for i in range(GK):
                pltpu.make_async_copy(
                    table_hbm.at[idx_v.at[g * GK + i]], rows_v.at[slot0 + i],
                    sem).wait()

        def start_out(qbase, g, slot0, sem):
            pltpu.async_copy(
                st_v.at[pl.ds(slot0, GK)],
                out_hbm.at[pl.ds(qbase + g * GK, GK), :], sem)

        def wait_out(qbase, g, slot0, sem):
            pltpu.make_async_copy(
                st_v.at[pl.ds(slot0, GK)],
                out_hbm.at[pl.ds(qbase + g * GK, GK), :], sem).wait()

        def reduce_group(slot0):
            for i in range(GK):
                slot = slot0 + i

                def red(r, accs, slot=slot):
                    out = list(accs)
                    for rr in range(RUN):
                        row = r * RUN + rr
                        for c in range(NCH):
                            out[c] = out[c] + rows_v[
                                slot, row, pl.ds(c * LANES, LANES)
                            ]
                    return tuple(out)

                accs = lax.fori_loop(
                    0, SEQ // RUN, red,
                    tuple(jnp.zeros((LANES,), jnp.float32) for _ in range(NCH)),
                )
                for c in range(NCH):
                    st_v[slot, pl.ds(c * LANES, LANES)] = accs[c]

        for q in range(DPW // QUART):
            qbase = base + q * QUART
            pltpu.sync_copy(tok_hbm.at[pl.ds(qbase, QUART), :], idx_v)
            fire(0, 0, sem_a)

            def jj_body(jj, carry, q=q, qbase=qbase):
                g = 2 * jj
                fire(g + 1, GK, sem_b)
                drain(g, 0, sem_a)

                @pl.when(jj > 0 if q == 0 else jj >= 0)
                def _():
                    wait_out(qbase, g, 0, soa)

                reduce_group(0)
                start_out(qbase, g, 0, soa)

                @pl.when(g + 2 < NG)
                def _():
                    fire(g + 2, 0, sem_a)

                drain(g + 1, GK, sem_b)

                @pl.when(jj > 0 if q == 0 else jj >= 0)
                def _():
                    wait_out(qbase, g + 1, GK, sob)

                reduce_group(GK)
                start_out(qbase, g + 1, GK, sob)
                return carry

            lax.fori_loop(0, NG // 2, jj_body, 0)

        wait_out(base, 0, 0, soa)
        wait_out(base, 0, GK, sob)

    return pool(tok2, table_lin.reshape(VOC2, DIM))


def _tc_proj(sums, W, b):
    blk = 2048

    def proj(s_ref, w_ref, b_ref, o_ref):
        o_ref[...] = (
            lax.dot_general(
                s_ref[...], w_ref[...], (((1,), (1,)), ((), ())),
                preferred_element_type=jnp.float32,
            ) * (1.0 / SEQ)
            + b_ref[...]
        )

    return pl.pallas_call(
        proj,
        grid=(NB // blk,),
        in_specs=[
            pl.BlockSpec((blk, DIM), lambda i: (i, 0)),
            pl.BlockSpec((DIM, DIM), lambda i: (0, 0)),
            pl.BlockSpec((1, DIM), lambda i: (0, 0)),
        ],
        out_specs=pl.BlockSpec((blk, DIM), lambda i: (i, 0)),
        out_shape=jax.ShapeDtypeStruct((NB, DIM), jnp.float32),
    )(sums, W, b.reshape(1, DIM))


@jax.jit
def kernel(token_ids, table, W, b):
    table_t = jnp.swapaxes(table, 0, 1)
    table_pairs = _tc_relayout(table_t)
    tok2 = jnp.where(token_ids < VSPLIT, 2 * token_ids,
                     2 * token_ids - (VOC2 - 1))
    sums = _sc_pool(tok2, table_pairs.reshape(VOC2 * DIM))
    return _tc_proj(sums, W, b)
